# Initial kernel scaffold; baseline (speedup 1.0000x reference)
import jax
import jax.numpy as jnp
from jax import lax
from jax.experimental import pallas as pl
from jax.experimental.pallas import tpu as pltpu
from jax.experimental.pallas import tpu_sc as plsc

N, E, D, DOM, MEM = 10000, 160000, 256, 4, 10
K = DOM * MEM
KP = 48
NC, NS, LANES = 2, 16, 16
NW = NC * NS
N_PAD = 10240
EC = 128
NCHUNK = 40
E_PAD = NW * NCHUNK * EC
ROWS_S = N_PAD // NS
TC_ROWS = 256

_mesh = plsc.VectorSubcoreMesh(core_axis_name="c", subcore_axis_name="s",
                               num_cores=NC, num_subcores=NS)


def _sc_deg_body(dst_hbm, deg_out, dst_v, ones_v, cbuf, acc):
    cid = lax.axis_index("c")
    sid = lax.axis_index("s")
    wid = sid * NC + cid

    ones16 = jnp.ones((LANES,), jnp.float32)
    zero16 = jnp.zeros((LANES,), jnp.float32)

    def fill_ones(i, carry):
        ones_v[i] = ones16
        return carry

    lax.fori_loop(0, EC, fill_ones, 0)

    def fill_zero(i, carry):
        cbuf[i] = zero16
        return carry

    lax.fori_loop(0, ROWS_S, fill_zero, 0)

    pltpu.sync_copy(cbuf, acc.at[pl.ds(sid * ROWS_S, ROWS_S)])
    plsc.subcore_barrier()

    pltpu.sync_copy(dst_hbm.at[wid], dst_v)

    def step(j, carry):
        pltpu.sync_copy(ones_v, acc.at[dst_v.at[j]], add=True)
        return carry

    lax.fori_loop(0, NCHUNK, step, 0)
    plsc.subcore_barrier()

    pltpu.sync_copy(acc.at[pl.ds(sid * ROWS_S, ROWS_S)], cbuf)
    pltpu.sync_copy(cbuf, deg_out.at[cid, pl.ds(sid * ROWS_S, ROWS_S)])


_sc_deg = pl.kernel(
    _sc_deg_body,
    out_type=jax.ShapeDtypeStruct((NC, N_PAD, LANES), jnp.float32),
    mesh=_mesh,
    scratch_types=[
        pltpu.VMEM((NCHUNK, EC), jnp.int32),
        pltpu.VMEM((EC, LANES), jnp.float32),
        pltpu.VMEM((ROWS_S, LANES), jnp.float32),
        pltpu.VMEM_SHARED((N_PAD, LANES), jnp.float32),
    ],
)


def _sc_agg_body(src_hbm, dst_hbm, z_hbm, y_out, src_v, dst_v, gbuf, zbuf,
                 acc, sem):
    cid = lax.axis_index("c")
    sid = lax.axis_index("s")
    wid = sid * NC + cid

    zero16 = jnp.zeros((LANES,), jnp.float32)

    def fill_zero(i, carry):
        zbuf[i, pl.ds(0, LANES)] = zero16
        zbuf[i, pl.ds(LANES, LANES)] = zero16
        zbuf[i, pl.ds(2 * LANES, LANES)] = zero16
        return carry

    lax.fori_loop(0, ROWS_S, fill_zero, 0)

    pltpu.sync_copy(zbuf, acc.at[pl.ds(sid * ROWS_S, ROWS_S)])
    plsc.subcore_barrier()

    pltpu.sync_copy(src_hbm.at[wid], src_v)
    pltpu.sync_copy(dst_hbm.at[wid], dst_v)

    def step(j, carry):
        pltpu.async_copy(z_hbm.at[src_v.at[j]], gbuf, sem).wait()
        pltpu.sync_copy(gbuf, acc.at[dst_v.at[j]], add=True)
        return carry

    lax.fori_loop(0, NCHUNK, step, 0)
    plsc.subcore_barrier()

    pltpu.sync_copy(acc.at[pl.ds(sid * ROWS_S, ROWS_S)], zbuf)
    pltpu.sync_copy(zbuf, y_out.at[cid, pl.ds(sid * ROWS_S, ROWS_S)])


_sc_agg = pl.kernel(
    _sc_agg_body,
    out_type=jax.ShapeDtypeStruct((NC, N_PAD, KP), jnp.float32),
    mesh=_mesh,
    scratch_types=[
        pltpu.VMEM((NCHUNK, EC), jnp.int32),
        pltpu.VMEM((NCHUNK, EC), jnp.int32),
        pltpu.VMEM((EC, KP), jnp.float32),
        pltpu.VMEM((ROWS_S, KP), jnp.float32),
        pltpu.VMEM_SHARED((N_PAD, KP), jnp.float32),
        pltpu.SemaphoreType.DMA,
    ],
)


def _tc_dense_body(f_ref, w_ref, m_ref, dp_ref, z_ref, t_ref):
    f = f_ref[...]
    nrm = jnp.sqrt(jnp.sum(f * f, axis=1, keepdims=True))
    feat = f / (nrm + 1e-12)
    h = jnp.dot(feat, w_ref[...], preferred_element_type=jnp.float32)
    hm = jnp.dot(h, m_ref[...], preferred_element_type=jnp.float32)
    tm = jnp.dot(feat, m_ref[...], preferred_element_type=jnp.float32)
    dp = dp_ref[...]
    deg = 1.0 + dp[0, :, 0:1] + dp[1, :, 0:1]
    dinv = lax.rsqrt(deg)
    z_ref[...] = dinv * hm
    t_ref[...] = tm


_tc_dense = pl.pallas_call(
    _tc_dense_body,
    grid=(N_PAD // TC_ROWS,),
    in_specs=[
        pl.BlockSpec((TC_ROWS, D), lambda i: (i, 0)),
        pl.BlockSpec((D, D), lambda i: (0, 0)),
        pl.BlockSpec((D, KP), lambda i: (0, 0)),
        pl.BlockSpec((NC, TC_ROWS, LANES), lambda i: (0, i, 0)),
    ],
    out_specs=[
        pl.BlockSpec((TC_ROWS, KP), lambda i: (i, 0)),
        pl.BlockSpec((TC_ROWS, KP), lambda i: (i, 0)),
    ],
    out_shape=[
        jax.ShapeDtypeStruct((N_PAD, KP), jnp.float32),
        jax.ShapeDtypeStruct((N_PAD, KP), jnp.float32),
    ],
)


def _tc_final_body(yp_ref, z_ref, t_ref, dp_ref, b_ref, m_ref, o_ref):
    yp = yp_ref[...]
    y = yp[0] + yp[1] + z_ref[...]
    dp = dp_ref[...]
    deg = 1.0 + dp[0, :, 0:1] + dp[1, :, 0:1]
    dinv = lax.rsqrt(deg)
    bm = jnp.dot(b_ref[...], m_ref[...], preferred_element_type=jnp.float32)
    s = dinv * y + bm
    t = t_ref[...]
    logits = []
    for d in range(DOM):
        sd = s[:, d * MEM:(d + 1) * MEM]
        td = t[:, d * MEM:(d + 1) * MEM]
        mx = jnp.max(sd, axis=1, keepdims=True)
        e = jnp.exp(sd - mx)
        num = jnp.sum(e * td, axis=1, keepdims=True)
        den = jnp.sum(e, axis=1, keepdims=True)
        logits.append(num / den)
    lg = jnp.concatenate(logits, axis=1)
    mm = jnp.max(lg, axis=1, keepdims=True)
    ee = jnp.exp(lg - mm)
    o_ref[...] = ee / jnp.sum(ee, axis=1, keepdims=True)


_tc_final = pl.pallas_call(
    _tc_final_body,
    grid=(N_PAD // TC_ROWS,),
    in_specs=[
        pl.BlockSpec((NC, TC_ROWS, KP), lambda i: (0, i, 0)),
        pl.BlockSpec((TC_ROWS, KP), lambda i: (i, 0)),
        pl.BlockSpec((TC_ROWS, KP), lambda i: (i, 0)),
        pl.BlockSpec((NC, TC_ROWS, LANES), lambda i: (0, i, 0)),
        pl.BlockSpec((1, D), lambda i: (0, 0)),
        pl.BlockSpec((D, KP), lambda i: (0, 0)),
    ],
    out_specs=pl.BlockSpec((TC_ROWS, DOM), lambda i: (i, 0)),
    out_shape=jax.ShapeDtypeStruct((N_PAD, DOM), jnp.float32),
)


def kernel(feature, category, edge_index, W, b, domain_memory):
    del category
    ei = edge_index.astype(jnp.int32)
    pad = jnp.full((E_PAD - E,), N, jnp.int32)
    src3 = jnp.concatenate([ei[0], pad]).reshape(NW, NCHUNK, EC)
    dst3 = jnp.concatenate([ei[1], pad]).reshape(NW, NCHUNK, EC)

    feature_pad = jnp.pad(feature, ((0, N_PAD - N), (0, 0)))
    mflat = domain_memory.reshape(K, D)
    mpad = jnp.zeros((D, KP), jnp.float32).at[:, :K].set(mflat.T)
    b_mat = b.reshape(1, D)

    deg_parts = _sc_deg(dst3)
    z, t = _tc_dense(feature_pad, W, mpad, deg_parts)
    y_parts = _sc_agg(src3, dst3, z)
    out_full = _tc_final(y_parts, z, t, deg_parts, b_mat, mpad)
    return out_full[:N, None, :]


# speedup vs baseline: 14.5914x; 14.5914x over previous
import jax
import jax.numpy as jnp
from jax import lax
from jax.experimental import pallas as pl
from jax.experimental.pallas import tpu as pltpu
from jax.experimental.pallas import tpu_sc as plsc

N, E, D, DOM, MEM = 10000, 160000, 256, 4, 10
K = DOM * MEM
KP = 48
NC, NS, LANES = 2, 16, 16
NW = NC * NS
N_PAD = 10240
EC = 128
NCHUNK = 40
E_PAD = NW * NCHUNK * EC
ROWS_S = N_PAD // NS
TC_ROWS = 256

_mesh = plsc.VectorSubcoreMesh(core_axis_name="c", subcore_axis_name="s",
                               num_cores=NC, num_subcores=NS)


def _sc_deg_body(dst_hbm, deg_out, dst_v, ones_v, cbuf, acc):
    cid = lax.axis_index("c")
    sid = lax.axis_index("s")
    wid = sid * NC + cid

    ones16 = jnp.ones((LANES,), jnp.float32)
    zero16 = jnp.zeros((LANES,), jnp.float32)

    def fill_ones(i, carry):
        ones_v[i] = ones16
        return carry

    lax.fori_loop(0, EC, fill_ones, 0)

    def fill_zero(i, carry):
        cbuf[i] = zero16
        return carry

    lax.fori_loop(0, ROWS_S, fill_zero, 0)

    pltpu.sync_copy(cbuf, acc.at[pl.ds(sid * ROWS_S, ROWS_S)])
    plsc.subcore_barrier()

    pltpu.sync_copy(dst_hbm.at[wid], dst_v)

    def step(j, carry):
        pltpu.sync_copy(ones_v, acc.at[dst_v.at[j]], add=True)
        return carry

    lax.fori_loop(0, NCHUNK, step, 0)
    plsc.subcore_barrier()

    pltpu.sync_copy(acc.at[pl.ds(sid * ROWS_S, ROWS_S)], cbuf)
    pltpu.sync_copy(cbuf, deg_out.at[cid, pl.ds(sid * ROWS_S, ROWS_S)])


_sc_deg = pl.kernel(
    _sc_deg_body,
    out_type=jax.ShapeDtypeStruct((NC, N_PAD, LANES), jnp.float32),
    mesh=_mesh,
    scratch_types=[
        pltpu.VMEM((NCHUNK, EC), jnp.int32),
        pltpu.VMEM((EC, LANES), jnp.float32),
        pltpu.VMEM((ROWS_S, LANES), jnp.float32),
        pltpu.VMEM_SHARED((N_PAD, LANES), jnp.float32),
    ],
    compiler_params=pltpu.CompilerParams(use_tc_tiling_on_sc=False),
)


def _sc_agg_body(src_hbm, dst_hbm, z_hbm, y_out, src_v, dst_v, gbuf, zbuf,
                 acc, sem):
    cid = lax.axis_index("c")
    sid = lax.axis_index("s")
    wid = sid * NC + cid

    zero16 = jnp.zeros((LANES,), jnp.float32)

    def fill_zero(i, carry):
        zbuf[i, pl.ds(0, LANES)] = zero16
        zbuf[i, pl.ds(LANES, LANES)] = zero16
        zbuf[i, pl.ds(2 * LANES, LANES)] = zero16
        return carry

    lax.fori_loop(0, ROWS_S, fill_zero, 0)

    pltpu.sync_copy(zbuf, acc.at[pl.ds(sid * ROWS_S, ROWS_S)])
    plsc.subcore_barrier()

    pltpu.sync_copy(src_hbm.at[wid], src_v)
    pltpu.sync_copy(dst_hbm.at[wid], dst_v)

    def step(j, carry):
        pltpu.async_copy(z_hbm.at[src_v.at[j]], gbuf, sem).wait()
        pltpu.sync_copy(gbuf, acc.at[dst_v.at[j]], add=True)
        return carry

    lax.fori_loop(0, NCHUNK, step, 0)
    plsc.subcore_barrier()

    pltpu.sync_copy(acc.at[pl.ds(sid * ROWS_S, ROWS_S)], zbuf)
    pltpu.sync_copy(zbuf, y_out.at[cid, pl.ds(sid * ROWS_S, ROWS_S)])


_sc_agg = pl.kernel(
    _sc_agg_body,
    out_type=jax.ShapeDtypeStruct((NC, N_PAD, KP), jnp.float32),
    mesh=_mesh,
    scratch_types=[
        pltpu.VMEM((NCHUNK, EC), jnp.int32),
        pltpu.VMEM((NCHUNK, EC), jnp.int32),
        pltpu.VMEM((EC, KP), jnp.float32),
        pltpu.VMEM((ROWS_S, KP), jnp.float32),
        pltpu.VMEM_SHARED((N_PAD, KP), jnp.float32),
        pltpu.SemaphoreType.DMA,
    ],
    compiler_params=pltpu.CompilerParams(use_tc_tiling_on_sc=False),
)


def _tc_dense_body(f_ref, w_ref, m_ref, dp_ref, z_ref, t_ref):
    f = f_ref[...]
    nrm = jnp.sqrt(jnp.sum(f * f, axis=1, keepdims=True))
    feat = f / (nrm + 1e-12)
    h = jnp.dot(feat, w_ref[...], preferred_element_type=jnp.float32)
    hm = jnp.dot(h, m_ref[...], preferred_element_type=jnp.float32)
    tm = jnp.dot(feat, m_ref[...], preferred_element_type=jnp.float32)
    dp = dp_ref[...]
    deg = 1.0 + dp[0, :, 0:1] + dp[1, :, 0:1]
    dinv = lax.rsqrt(deg)
    z_ref[...] = dinv * hm
    t_ref[...] = tm


_tc_dense = pl.pallas_call(
    _tc_dense_body,
    grid=(N_PAD // TC_ROWS,),
    in_specs=[
        pl.BlockSpec((TC_ROWS, D), lambda i: (i, 0)),
        pl.BlockSpec((D, D), lambda i: (0, 0)),
        pl.BlockSpec((D, KP), lambda i: (0, 0)),
        pl.BlockSpec((NC, TC_ROWS, LANES), lambda i: (0, i, 0)),
    ],
    out_specs=[
        pl.BlockSpec((TC_ROWS, KP), lambda i: (i, 0)),
        pl.BlockSpec((TC_ROWS, KP), lambda i: (i, 0)),
    ],
    out_shape=[
        jax.ShapeDtypeStruct((N_PAD, KP), jnp.float32),
        jax.ShapeDtypeStruct((N_PAD, KP), jnp.float32),
    ],
)


def _tc_final_body(yp_ref, z_ref, t_ref, dp_ref, b_ref, m_ref, o_ref):
    yp = yp_ref[...]
    y = yp[0] + yp[1] + z_ref[...]
    dp = dp_ref[...]
    deg = 1.0 + dp[0, :, 0:1] + dp[1, :, 0:1]
    dinv = lax.rsqrt(deg)
    bm = jnp.dot(b_ref[...], m_ref[...], preferred_element_type=jnp.float32)
    s = dinv * y + bm
    t = t_ref[...]
    logits = []
    for d in range(DOM):
        sd = s[:, d * MEM:(d + 1) * MEM]
        td = t[:, d * MEM:(d + 1) * MEM]
        mx = jnp.max(sd, axis=1, keepdims=True)
        e = jnp.exp(sd - mx)
        num = jnp.sum(e * td, axis=1, keepdims=True)
        den = jnp.sum(e, axis=1, keepdims=True)
        logits.append(num / den)
    lg = jnp.concatenate(logits, axis=1)
    mm = jnp.max(lg, axis=1, keepdims=True)
    ee = jnp.exp(lg - mm)
    o_ref[...] = ee / jnp.sum(ee, axis=1, keepdims=True)


_tc_final = pl.pallas_call(
    _tc_final_body,
    grid=(N_PAD // TC_ROWS,),
    in_specs=[
        pl.BlockSpec((NC, TC_ROWS, KP), lambda i: (0, i, 0)),
        pl.BlockSpec((TC_ROWS, KP), lambda i: (i, 0)),
        pl.BlockSpec((TC_ROWS, KP), lambda i: (i, 0)),
        pl.BlockSpec((NC, TC_ROWS, LANES), lambda i: (0, i, 0)),
        pl.BlockSpec((1, D), lambda i: (0, 0)),
        pl.BlockSpec((D, KP), lambda i: (0, 0)),
    ],
    out_specs=pl.BlockSpec((TC_ROWS, DOM), lambda i: (i, 0)),
    out_shape=jax.ShapeDtypeStruct((N_PAD, DOM), jnp.float32),
)


def kernel(feature, category, edge_index, W, b, domain_memory):
    del category
    ei = edge_index.astype(jnp.int32)
    pad = jnp.full((E_PAD - E,), N, jnp.int32)
    src3 = jnp.concatenate([ei[0], pad]).reshape(NW, NCHUNK, EC)
    dst3 = jnp.concatenate([ei[1], pad]).reshape(NW, NCHUNK, EC)

    feature_pad = jnp.pad(feature, ((0, N_PAD - N), (0, 0)))
    mflat = domain_memory.reshape(K, D)
    mpad = jnp.zeros((D, KP), jnp.float32).at[:, :K].set(mflat.T)
    b_mat = b.reshape(1, D)

    deg_parts = _sc_deg(dst3)
    z, t = _tc_dense(feature_pad, W, mpad, deg_parts)
    y_parts = _sc_agg(src3, dst3, z)
    out_full = _tc_final(y_parts, z, t, deg_parts, b_mat, mpad)
    return out_full[:N, None, :]


# speedup vs baseline: 15.8124x; 1.0837x over previous
import jax
import jax.numpy as jnp
from jax import lax
from jax.experimental import pallas as pl
from jax.experimental.pallas import tpu as pltpu
from jax.experimental.pallas import tpu_sc as plsc

N, E, D, DOM, MEM = 10000, 160000, 256, 4, 10
K = DOM * MEM
KP = 48
NC, NS, LANES = 2, 16, 16
NW = NC * NS
N_PAD = 10240
EC = 128
NCHUNK = 40
E_PAD = NW * NCHUNK * EC
ROWS_S = N_PAD // NS
TC_ROWS = 256

_mesh = plsc.VectorSubcoreMesh(core_axis_name="c", subcore_axis_name="s",
                               num_cores=NC, num_subcores=NS)


def _sc_deg_body(dst_hbm, deg_out, dst_v, ones_v, cbuf, acc):
    cid = lax.axis_index("c")
    sid = lax.axis_index("s")
    wid = sid * NC + cid

    ones16 = jnp.ones((LANES,), jnp.float32)
    zero16 = jnp.zeros((LANES,), jnp.float32)

    def fill_ones(i, carry):
        ones_v[i] = ones16
        return carry

    lax.fori_loop(0, EC, fill_ones, 0)

    def fill_zero(i, carry):
        cbuf[i] = zero16
        return carry

    lax.fori_loop(0, ROWS_S, fill_zero, 0)

    pltpu.sync_copy(cbuf, acc.at[pl.ds(sid * ROWS_S, ROWS_S)])
    plsc.subcore_barrier()

    pltpu.sync_copy(dst_hbm.at[wid], dst_v)

    def step(j, carry):
        pltpu.sync_copy(ones_v, acc.at[dst_v.at[j]], add=True)
        return carry

    lax.fori_loop(0, NCHUNK, step, 0)
    plsc.subcore_barrier()

    pltpu.sync_copy(acc.at[pl.ds(sid * ROWS_S, ROWS_S)], cbuf)
    pltpu.sync_copy(cbuf, deg_out.at[cid, pl.ds(sid * ROWS_S, ROWS_S)])


_sc_deg = pl.kernel(
    _sc_deg_body,
    out_type=jax.ShapeDtypeStruct((NC, N_PAD, LANES), jnp.float32),
    mesh=_mesh,
    scratch_types=[
        pltpu.VMEM((NCHUNK, EC), jnp.int32),
        pltpu.VMEM((EC, LANES), jnp.float32),
        pltpu.VMEM((ROWS_S, LANES), jnp.float32),
        pltpu.VMEM_SHARED((N_PAD, LANES), jnp.float32),
    ],
    compiler_params=pltpu.CompilerParams(use_tc_tiling_on_sc=False),
)


NB = 4


def _sc_agg_body(src_hbm, dst_hbm, z_hbm, y_out, src_v, dst_v, gb0, gb1, gb2,
                 gb3, zbuf, acc, sem0, sem1, sem2, sem3):
    cid = lax.axis_index("c")
    sid = lax.axis_index("s")
    wid = sid * NC + cid
    gbufs = (gb0, gb1, gb2, gb3)
    sems = (sem0, sem1, sem2, sem3)

    zero16 = jnp.zeros((LANES,), jnp.float32)

    def fill_zero(i, carry):
        zbuf[i, pl.ds(0, LANES)] = zero16
        zbuf[i, pl.ds(LANES, LANES)] = zero16
        zbuf[i, pl.ds(2 * LANES, LANES)] = zero16
        return carry

    lax.fori_loop(0, ROWS_S, fill_zero, 0)

    pltpu.sync_copy(zbuf, acc.at[pl.ds(sid * ROWS_S, ROWS_S)])

    pltpu.sync_copy(src_hbm.at[wid], src_v)
    pltpu.sync_copy(dst_hbm.at[wid], dst_v)
    plsc.subcore_barrier()

    for b in range(NB):
        pltpu.async_copy(z_hbm.at[src_v.at[b]], gbufs[b], sems[b])

    def step(g, carry):
        for b in range(NB):
            j = NB * g + b
            pltpu.make_async_copy(z_hbm.at[src_v.at[j]], gbufs[b],
                                  sems[b]).wait()
            pltpu.sync_copy(gbufs[b], acc.at[dst_v.at[j]], add=True)
            nxt = j + NB

            @pl.when(nxt < NCHUNK)
            def _():
                pltpu.async_copy(z_hbm.at[src_v.at[nxt]], gbufs[b], sems[b])

        return carry

    lax.fori_loop(0, NCHUNK // NB, step, 0)
    plsc.subcore_barrier()

    pltpu.sync_copy(acc.at[pl.ds(sid * ROWS_S, ROWS_S)], zbuf)
    pltpu.sync_copy(zbuf, y_out.at[cid, pl.ds(sid * ROWS_S, ROWS_S)])


_sc_agg = pl.kernel(
    _sc_agg_body,
    out_type=jax.ShapeDtypeStruct((NC, N_PAD, KP), jnp.float32),
    mesh=_mesh,
    scratch_types=[
        pltpu.VMEM((NCHUNK, EC), jnp.int32),
        pltpu.VMEM((NCHUNK, EC), jnp.int32),
        pltpu.VMEM((EC, KP), jnp.float32),
        pltpu.VMEM((EC, KP), jnp.float32),
        pltpu.VMEM((EC, KP), jnp.float32),
        pltpu.VMEM((EC, KP), jnp.float32),
        pltpu.VMEM((ROWS_S, KP), jnp.float32),
        pltpu.VMEM_SHARED((N_PAD, KP), jnp.float32),
        pltpu.SemaphoreType.DMA,
        pltpu.SemaphoreType.DMA,
        pltpu.SemaphoreType.DMA,
        pltpu.SemaphoreType.DMA,
    ],
    compiler_params=pltpu.CompilerParams(use_tc_tiling_on_sc=False),
)


def _tc_dense_body(f_ref, w_ref, m_ref, dp_ref, z_ref, t_ref):
    f = f_ref[...]
    nrm = jnp.sqrt(jnp.sum(f * f, axis=1, keepdims=True))
    feat = f / (nrm + 1e-12)
    h = jnp.dot(feat, w_ref[...], preferred_element_type=jnp.float32)
    hm = jnp.dot(h, m_ref[...], preferred_element_type=jnp.float32)
    tm = jnp.dot(feat, m_ref[...], preferred_element_type=jnp.float32)
    dp = dp_ref[...]
    deg = 1.0 + dp[0, :, 0:1] + dp[1, :, 0:1]
    dinv = lax.rsqrt(deg)
    z_ref[...] = dinv * hm
    t_ref[...] = tm


_tc_dense = pl.pallas_call(
    _tc_dense_body,
    grid=(N_PAD // TC_ROWS,),
    in_specs=[
        pl.BlockSpec((TC_ROWS, D), lambda i: (i, 0)),
        pl.BlockSpec((D, D), lambda i: (0, 0)),
        pl.BlockSpec((D, KP), lambda i: (0, 0)),
        pl.BlockSpec((NC, TC_ROWS, LANES), lambda i: (0, i, 0)),
    ],
    out_specs=[
        pl.BlockSpec((TC_ROWS, KP), lambda i: (i, 0)),
        pl.BlockSpec((TC_ROWS, KP), lambda i: (i, 0)),
    ],
    out_shape=[
        jax.ShapeDtypeStruct((N_PAD, KP), jnp.float32),
        jax.ShapeDtypeStruct((N_PAD, KP), jnp.float32),
    ],
)


def _tc_final_body(yp_ref, z_ref, t_ref, dp_ref, b_ref, m_ref, o_ref):
    yp = yp_ref[...]
    y = yp[0] + yp[1] + z_ref[...]
    dp = dp_ref[...]
    deg = 1.0 + dp[0, :, 0:1] + dp[1, :, 0:1]
    dinv = lax.rsqrt(deg)
    bm = jnp.dot(b_ref[...], m_ref[...], preferred_element_type=jnp.float32)
    s = dinv * y + bm
    t = t_ref[...]
    logits = []
    for d in range(DOM):
        sd = s[:, d * MEM:(d + 1) * MEM]
        td = t[:, d * MEM:(d + 1) * MEM]
        mx = jnp.max(sd, axis=1, keepdims=True)
        e = jnp.exp(sd - mx)
        num = jnp.sum(e * td, axis=1, keepdims=True)
        den = jnp.sum(e, axis=1, keepdims=True)
        logits.append(num / den)
    lg = jnp.concatenate(logits, axis=1)
    mm = jnp.max(lg, axis=1, keepdims=True)
    ee = jnp.exp(lg - mm)
    o_ref[...] = ee / jnp.sum(ee, axis=1, keepdims=True)


_tc_final = pl.pallas_call(
    _tc_final_body,
    grid=(N_PAD // TC_ROWS,),
    in_specs=[
        pl.BlockSpec((NC, TC_ROWS, KP), lambda i: (0, i, 0)),
        pl.BlockSpec((TC_ROWS, KP), lambda i: (i, 0)),
        pl.BlockSpec((TC_ROWS, KP), lambda i: (i, 0)),
        pl.BlockSpec((NC, TC_ROWS, LANES), lambda i: (0, i, 0)),
        pl.BlockSpec((1, D), lambda i: (0, 0)),
        pl.BlockSpec((D, KP), lambda i: (0, 0)),
    ],
    out_specs=pl.BlockSpec((TC_ROWS, DOM), lambda i: (i, 0)),
    out_shape=jax.ShapeDtypeStruct((N_PAD, DOM), jnp.float32),
)


def kernel(feature, category, edge_index, W, b, domain_memory):
    del category
    ei = edge_index.astype(jnp.int32)
    pad = jnp.full((E_PAD - E,), N, jnp.int32)
    src3 = jnp.concatenate([ei[0], pad]).reshape(NW, NCHUNK, EC)
    dst3 = jnp.concatenate([ei[1], pad]).reshape(NW, NCHUNK, EC)

    feature_pad = jnp.pad(feature, ((0, N_PAD - N), (0, 0)))
    mflat = domain_memory.reshape(K, D)
    mpad = jnp.zeros((D, KP), jnp.float32).at[:, :K].set(mflat.T)
    b_mat = b.reshape(1, D)

    deg_parts = _sc_deg(dst3)
    z, t = _tc_dense(feature_pad, W, mpad, deg_parts)
    y_parts = _sc_agg(src3, dst3, z)
    out_full = _tc_final(y_parts, z, t, deg_parts, b_mat, mpad)
    return out_full[:N, None, :]


# speedup vs baseline: 18.8102x; 1.1896x over previous
import jax
import jax.numpy as jnp
import numpy as np
from jax import lax
from jax.experimental import pallas as pl
from jax.experimental.pallas import tpu as pltpu
from jax.experimental.pallas import tpu_sc as plsc

N, E, D, DOM, MEM = 10000, 160000, 256, 4, 10
K = DOM * MEM
KP = 48
NC, NS, LANES = 2, 16, 16
NW = NC * NS
N_PAD = 10240
EC = 128
NCHUNK = 40
E_PAD = NW * NCHUNK * EC
ROWS_S = N_PAD // NS
TC_ROWS = 512

_mesh = plsc.VectorSubcoreMesh(core_axis_name="c", subcore_axis_name="s",
                               num_cores=NC, num_subcores=NS)


def _sc_deg_body(dst_hbm, deg_out, dst_v, ones_v, cbuf, acc):
    cid = lax.axis_index("c")
    sid = lax.axis_index("s")
    wid = sid * NC + cid

    ones16 = jnp.ones((LANES,), jnp.float32)
    zero16 = jnp.zeros((LANES,), jnp.float32)

    def fill_ones(i, carry):
        ones_v[i] = ones16
        return carry

    lax.fori_loop(0, EC, fill_ones, 0)

    def fill_zero(i, carry):
        cbuf[i] = zero16
        return carry

    lax.fori_loop(0, ROWS_S, fill_zero, 0)

    pltpu.sync_copy(cbuf, acc.at[pl.ds(sid * ROWS_S, ROWS_S)])
    plsc.subcore_barrier()

    pltpu.sync_copy(dst_hbm.at[wid], dst_v)

    def step(j, carry):
        pltpu.sync_copy(ones_v, acc.at[dst_v.at[j]], add=True)
        return carry

    lax.fori_loop(0, NCHUNK, step, 0)
    plsc.subcore_barrier()

    pltpu.sync_copy(acc.at[pl.ds(sid * ROWS_S, ROWS_S)], cbuf)
    pltpu.sync_copy(cbuf, deg_out.at[cid, pl.ds(sid * ROWS_S, ROWS_S)])


_sc_deg = pl.kernel(
    _sc_deg_body,
    out_type=jax.ShapeDtypeStruct((NC, N_PAD, LANES), jnp.float32),
    mesh=_mesh,
    scratch_types=[
        pltpu.VMEM((NCHUNK, EC), jnp.int32),
        pltpu.VMEM((EC, LANES), jnp.float32),
        pltpu.VMEM((ROWS_S, LANES), jnp.float32),
        pltpu.VMEM_SHARED((N_PAD, LANES), jnp.float32),
    ],
    compiler_params=pltpu.CompilerParams(use_tc_tiling_on_sc=False),
)


NB = 4


def _sc_agg_body(src_hbm, dst_hbm, z_hbm, y_out, src_v, dst_v, gb0, gb1, gb2,
                 gb3, zbuf, acc, sem0, sem1, sem2, sem3):
    cid = lax.axis_index("c")
    sid = lax.axis_index("s")
    wid = sid * NC + cid
    gbufs = (gb0, gb1, gb2, gb3)
    sems = (sem0, sem1, sem2, sem3)

    zero16 = jnp.zeros((LANES,), jnp.float32)

    def fill_zero(i, carry):
        zbuf[i, pl.ds(0, LANES)] = zero16
        zbuf[i, pl.ds(LANES, LANES)] = zero16
        zbuf[i, pl.ds(2 * LANES, LANES)] = zero16
        return carry

    lax.fori_loop(0, ROWS_S, fill_zero, 0)

    pltpu.sync_copy(zbuf, acc.at[pl.ds(sid * ROWS_S, ROWS_S)])

    pltpu.sync_copy(src_hbm.at[wid], src_v)
    pltpu.sync_copy(dst_hbm.at[wid], dst_v)
    plsc.subcore_barrier()

    for b in range(NB):
        pltpu.async_copy(z_hbm.at[src_v.at[b]], gbufs[b], sems[b])

    def step(g, carry):
        for b in range(NB):
            j = NB * g + b
            pltpu.make_async_copy(z_hbm.at[src_v.at[j]], gbufs[b],
                                  sems[b]).wait()
            pltpu.sync_copy(gbufs[b], acc.at[dst_v.at[j]], add=True)
            nxt = j + NB

            @pl.when(nxt < NCHUNK)
            def _():
                pltpu.async_copy(z_hbm.at[src_v.at[nxt]], gbufs[b], sems[b])

        return carry

    lax.fori_loop(0, NCHUNK // NB, step, 0)
    plsc.subcore_barrier()

    pltpu.sync_copy(acc.at[pl.ds(sid * ROWS_S, ROWS_S)], zbuf)
    pltpu.sync_copy(zbuf, y_out.at[cid, pl.ds(sid * ROWS_S, ROWS_S)])


_sc_agg = pl.kernel(
    _sc_agg_body,
    out_type=jax.ShapeDtypeStruct((NC, N_PAD, KP), jnp.float32),
    mesh=_mesh,
    scratch_types=[
        pltpu.VMEM((NCHUNK, EC), jnp.int32),
        pltpu.VMEM((NCHUNK, EC), jnp.int32),
        pltpu.VMEM((EC, KP), jnp.float32),
        pltpu.VMEM((EC, KP), jnp.float32),
        pltpu.VMEM((EC, KP), jnp.float32),
        pltpu.VMEM((EC, KP), jnp.float32),
        pltpu.VMEM((ROWS_S, KP), jnp.float32),
        pltpu.VMEM_SHARED((N_PAD, KP), jnp.float32),
        pltpu.SemaphoreType.DMA,
        pltpu.SemaphoreType.DMA,
        pltpu.SemaphoreType.DMA,
        pltpu.SemaphoreType.DMA,
    ],
    compiler_params=pltpu.CompilerParams(use_tc_tiling_on_sc=False),
)


def _tc_dense_body(f_ref, w_ref, m_ref, dp_ref, z_ref, t_ref):
    f = f_ref[...]
    nrm = jnp.sqrt(jnp.sum(f * f, axis=1, keepdims=True))
    feat = f / (nrm + 1e-12)
    h = jnp.dot(feat, w_ref[...], preferred_element_type=jnp.float32)
    hm = jnp.dot(h, m_ref[...], preferred_element_type=jnp.float32)
    tm = jnp.dot(feat, m_ref[...], preferred_element_type=jnp.float32)
    dp = dp_ref[...]
    deg = 1.0 + dp[0, :, 0:1] + dp[1, :, 0:1]
    dinv = lax.rsqrt(deg)
    z_ref[...] = dinv * hm
    t_ref[...] = tm


_tc_dense = pl.pallas_call(
    _tc_dense_body,
    grid=(N_PAD // TC_ROWS,),
    in_specs=[
        pl.BlockSpec((TC_ROWS, D), lambda i: (i, 0)),
        pl.BlockSpec((D, D), lambda i: (0, 0)),
        pl.BlockSpec((D, KP), lambda i: (0, 0)),
        pl.BlockSpec((NC, TC_ROWS, LANES), lambda i: (0, i, 0)),
    ],
    out_specs=[
        pl.BlockSpec((TC_ROWS, KP), lambda i: (i, 0)),
        pl.BlockSpec((TC_ROWS, KP), lambda i: (i, 0)),
    ],
    out_shape=[
        jax.ShapeDtypeStruct((N_PAD, KP), jnp.float32),
        jax.ShapeDtypeStruct((N_PAD, KP), jnp.float32),
    ],
)


def _tc_final_body(yp_ref, z_ref, t_ref, dp_ref, b_ref, m_ref, o_ref):
    rows = lax.broadcasted_iota(jnp.int32, (KP, DOM), 0)
    cols = lax.broadcasted_iota(jnp.int32, (KP, DOM), 1)
    seg = jnp.where((rows // MEM == cols) & (rows < K), 1.0, 0.0)
    rows2 = lax.broadcasted_iota(jnp.int32, (DOM, KP), 0)
    cols2 = lax.broadcasted_iota(jnp.int32, (DOM, KP), 1)
    segt = jnp.where((cols2 // MEM == rows2) & (cols2 < K), 1.0, 0.0)
    mask = (lax.broadcasted_iota(jnp.int32, (1, KP), 1) < K).astype(
        jnp.float32)
    yp = yp_ref[...]
    y = yp[0] + yp[1] + z_ref[...]
    dp = dp_ref[...]
    deg = 1.0 + dp[0, :, 0:1] + dp[1, :, 0:1]
    dinv = lax.rsqrt(deg)
    bm = jnp.dot(b_ref[...], m_ref[...], preferred_element_type=jnp.float32)
    s = dinv * y + bm
    t = t_ref[...]
    mx4 = jnp.concatenate(
        [jnp.max(s[:, d * MEM:(d + 1) * MEM], axis=1, keepdims=True)
         for d in range(DOM)], axis=1)
    sub = s - jnp.dot(mx4, segt, preferred_element_type=jnp.float32)
    e = jnp.exp(sub) * mask
    den4 = jnp.dot(e, seg, preferred_element_type=jnp.float32)
    num4 = jnp.dot(e * t, seg, preferred_element_type=jnp.float32)
    lg = num4 / den4
    mm = jnp.max(lg, axis=1, keepdims=True)
    ee = jnp.exp(lg - mm)
    o_ref[...] = ee / jnp.sum(ee, axis=1, keepdims=True)


_tc_final = pl.pallas_call(
    _tc_final_body,
    grid=(N_PAD // TC_ROWS,),
    in_specs=[
        pl.BlockSpec((NC, TC_ROWS, KP), lambda i: (0, i, 0)),
        pl.BlockSpec((TC_ROWS, KP), lambda i: (i, 0)),
        pl.BlockSpec((TC_ROWS, KP), lambda i: (i, 0)),
        pl.BlockSpec((NC, TC_ROWS, LANES), lambda i: (0, i, 0)),
        pl.BlockSpec((1, D), lambda i: (0, 0)),
        pl.BlockSpec((D, KP), lambda i: (0, 0)),
    ],
    out_specs=pl.BlockSpec((TC_ROWS, DOM), lambda i: (i, 0)),
    out_shape=jax.ShapeDtypeStruct((N_PAD, DOM), jnp.float32),
)


def kernel(feature, category, edge_index, W, b, domain_memory):
    del category
    ei = edge_index.astype(jnp.int32)
    pad = jnp.full((E_PAD - E,), N, jnp.int32)
    src3 = jnp.concatenate([ei[0], pad]).reshape(NW, NCHUNK, EC)
    dst3 = jnp.concatenate([ei[1], pad]).reshape(NW, NCHUNK, EC)

    feature_pad = jnp.pad(feature, ((0, N_PAD - N), (0, 0)))
    mflat = domain_memory.reshape(K, D)
    mpad = jnp.zeros((D, KP), jnp.float32).at[:, :K].set(mflat.T)
    b_mat = b.reshape(1, D)

    deg_parts = _sc_deg(dst3)
    z, t = _tc_dense(feature_pad, W, mpad, deg_parts)
    y_parts = _sc_agg(src3, dst3, z)
    out_full = _tc_final(y_parts, z, t, deg_parts, b_mat, mpad)
    return out_full[:N, None, :]


# speedup vs baseline: 28.0850x; 1.4931x over previous
import jax
import jax.numpy as jnp
import numpy as np
from jax import lax
from jax.experimental import pallas as pl
from jax.experimental.pallas import tpu as pltpu
from jax.experimental.pallas import tpu_sc as plsc

N, E, D, DOM, MEM = 10000, 160000, 256, 4, 10
K = DOM * MEM
KP = 48
NC, NS, LANES = 2, 16, 16
NW = NC * NS
N_PAD = 10240
EC = 128
NCHUNK = 40
E_PAD = NW * NCHUNK * EC
ROWS_S = N_PAD // NS
TC_ROWS = 512

_mesh = plsc.VectorSubcoreMesh(core_axis_name="c", subcore_axis_name="s",
                               num_cores=NC, num_subcores=NS)


def _sc_deg_body(dst_hbm, deg_out, dst_v, ones_v, cbuf, acc):
    cid = lax.axis_index("c")
    sid = lax.axis_index("s")
    wid = sid * NC + cid

    ones16 = jnp.ones((LANES,), jnp.float32)
    zero16 = jnp.zeros((LANES,), jnp.float32)

    def fill_ones(i, carry):
        ones_v[i] = ones16
        return carry

    lax.fori_loop(0, EC, fill_ones, 0)

    def fill_zero(i, carry):
        cbuf[i] = zero16
        return carry

    lax.fori_loop(0, ROWS_S, fill_zero, 0)

    pltpu.sync_copy(cbuf, acc.at[pl.ds(sid * ROWS_S, ROWS_S)])
    plsc.subcore_barrier()

    pltpu.sync_copy(dst_hbm.at[wid], dst_v)

    def step(j, carry):
        pltpu.sync_copy(ones_v, acc.at[dst_v.at[j]], add=True)
        return carry

    lax.fori_loop(0, NCHUNK, step, 0)
    plsc.subcore_barrier()

    pltpu.sync_copy(acc.at[pl.ds(sid * ROWS_S, ROWS_S)], cbuf)
    pltpu.sync_copy(cbuf, deg_out.at[cid, pl.ds(sid * ROWS_S, ROWS_S)])


_sc_deg = pl.kernel(
    _sc_deg_body,
    out_type=jax.ShapeDtypeStruct((NC, N_PAD, LANES), jnp.float32),
    mesh=_mesh,
    scratch_types=[
        pltpu.VMEM((NCHUNK, EC), jnp.int32),
        pltpu.VMEM((EC, LANES), jnp.float32),
        pltpu.VMEM((ROWS_S, LANES), jnp.float32),
        pltpu.VMEM_SHARED((N_PAD, LANES), jnp.float32),
    ],
    compiler_params=pltpu.CompilerParams(use_tc_tiling_on_sc=False),
)


NB = 4


def _sc_agg_body(src_hbm, dst_hbm, z_hbm, y_out, src_v, dst_v, gb0, gb1, gb2,
                 gb3, zbuf, acc, zs, sem0, sem1, sem2, sem3):
    cid = lax.axis_index("c")
    sid = lax.axis_index("s")
    wid = sid * NC + cid
    gbufs = (gb0, gb1, gb2, gb3)
    sems = (sem0, sem1, sem2, sem3)

    zero16 = jnp.zeros((LANES,), jnp.float32)

    def fill_zero(i, carry):
        zbuf[i, pl.ds(0, LANES)] = zero16
        zbuf[i, pl.ds(LANES, LANES)] = zero16
        zbuf[i, pl.ds(2 * LANES, LANES)] = zero16
        return carry

    lax.fori_loop(0, ROWS_S, fill_zero, 0)

    pltpu.sync_copy(zbuf, acc.at[pl.ds(sid * ROWS_S, ROWS_S)])

    pltpu.sync_copy(z_hbm.at[pl.ds(sid * ROWS_S, ROWS_S)], zbuf)
    pltpu.sync_copy(zbuf, zs.at[pl.ds(sid * ROWS_S, ROWS_S)])

    pltpu.sync_copy(src_hbm.at[wid], src_v)
    pltpu.sync_copy(dst_hbm.at[wid], dst_v)
    plsc.subcore_barrier()

    for b in range(NB):
        pltpu.async_copy(zs.at[src_v.at[b]], gbufs[b], sems[b])

    def step(g, carry):
        for b in range(NB):
            j = NB * g + b
            pltpu.make_async_copy(zs.at[src_v.at[j]], gbufs[b],
                                  sems[b]).wait()
            pltpu.sync_copy(gbufs[b], acc.at[dst_v.at[j]], add=True)
            nxt = j + NB

            @pl.when(nxt < NCHUNK)
            def _():
                pltpu.async_copy(zs.at[src_v.at[nxt]], gbufs[b], sems[b])

        return carry

    lax.fori_loop(0, NCHUNK // NB, step, 0)
    plsc.subcore_barrier()

    pltpu.sync_copy(acc.at[pl.ds(sid * ROWS_S, ROWS_S)], zbuf)
    pltpu.sync_copy(zbuf, y_out.at[cid, pl.ds(sid * ROWS_S, ROWS_S)])


_sc_agg = pl.kernel(
    _sc_agg_body,
    out_type=jax.ShapeDtypeStruct((NC, N_PAD, KP), jnp.float32),
    mesh=_mesh,
    scratch_types=[
        pltpu.VMEM((NCHUNK, EC), jnp.int32),
        pltpu.VMEM((NCHUNK, EC), jnp.int32),
        pltpu.VMEM((EC, KP), jnp.float32),
        pltpu.VMEM((EC, KP), jnp.float32),
        pltpu.VMEM((EC, KP), jnp.float32),
        pltpu.VMEM((EC, KP), jnp.float32),
        pltpu.VMEM((ROWS_S, KP), jnp.float32),
        pltpu.VMEM_SHARED((N_PAD, KP), jnp.float32),
        pltpu.VMEM_SHARED((N_PAD, KP), jnp.float32),
        pltpu.SemaphoreType.DMA,
        pltpu.SemaphoreType.DMA,
        pltpu.SemaphoreType.DMA,
        pltpu.SemaphoreType.DMA,
    ],
    compiler_params=pltpu.CompilerParams(use_tc_tiling_on_sc=False),
)


def _tc_dense_body(f_ref, w_ref, m_ref, dp_ref, z_ref, t_ref):
    f = f_ref[...]
    nrm = jnp.sqrt(jnp.sum(f * f, axis=1, keepdims=True))
    feat = f / (nrm + 1e-12)
    h = jnp.dot(feat, w_ref[...], preferred_element_type=jnp.float32)
    hm = jnp.dot(h, m_ref[...], preferred_element_type=jnp.float32)
    tm = jnp.dot(feat, m_ref[...], preferred_element_type=jnp.float32)
    dp = dp_ref[...]
    deg = 1.0 + dp[0, :, 0:1] + dp[1, :, 0:1]
    dinv = lax.rsqrt(deg)
    z_ref[...] = dinv * hm
    t_ref[...] = tm


_tc_dense = pl.pallas_call(
    _tc_dense_body,
    grid=(N_PAD // TC_ROWS,),
    in_specs=[
        pl.BlockSpec((TC_ROWS, D), lambda i: (i, 0)),
        pl.BlockSpec((D, D), lambda i: (0, 0)),
        pl.BlockSpec((D, KP), lambda i: (0, 0)),
        pl.BlockSpec((NC, TC_ROWS, LANES), lambda i: (0, i, 0)),
    ],
    out_specs=[
        pl.BlockSpec((TC_ROWS, KP), lambda i: (i, 0)),
        pl.BlockSpec((TC_ROWS, KP), lambda i: (i, 0)),
    ],
    out_shape=[
        jax.ShapeDtypeStruct((N_PAD, KP), jnp.float32),
        jax.ShapeDtypeStruct((N_PAD, KP), jnp.float32),
    ],
)


def _tc_final_body(yp_ref, z_ref, t_ref, dp_ref, b_ref, m_ref, o_ref):
    rows = lax.broadcasted_iota(jnp.int32, (KP, DOM), 0)
    cols = lax.broadcasted_iota(jnp.int32, (KP, DOM), 1)
    seg = jnp.where((rows // MEM == cols) & (rows < K), 1.0, 0.0)
    rows2 = lax.broadcasted_iota(jnp.int32, (DOM, KP), 0)
    cols2 = lax.broadcasted_iota(jnp.int32, (DOM, KP), 1)
    segt = jnp.where((cols2 // MEM == rows2) & (cols2 < K), 1.0, 0.0)
    mask = (lax.broadcasted_iota(jnp.int32, (1, KP), 1) < K).astype(
        jnp.float32)
    yp = yp_ref[...]
    y = yp[0] + yp[1] + z_ref[...]
    dp = dp_ref[...]
    deg = 1.0 + dp[0, :, 0:1] + dp[1, :, 0:1]
    dinv = lax.rsqrt(deg)
    bm = jnp.dot(b_ref[...], m_ref[...], preferred_element_type=jnp.float32)
    s = dinv * y + bm
    t = t_ref[...]
    mx4 = jnp.concatenate(
        [jnp.max(s[:, d * MEM:(d + 1) * MEM], axis=1, keepdims=True)
         for d in range(DOM)], axis=1)
    sub = s - jnp.dot(mx4, segt, preferred_element_type=jnp.float32)
    e = jnp.exp(sub) * mask
    den4 = jnp.dot(e, seg, preferred_element_type=jnp.float32)
    num4 = jnp.dot(e * t, seg, preferred_element_type=jnp.float32)
    lg = num4 / den4
    mm = jnp.max(lg, axis=1, keepdims=True)
    ee = jnp.exp(lg - mm)
    o_ref[...] = ee / jnp.sum(ee, axis=1, keepdims=True)


_tc_final = pl.pallas_call(
    _tc_final_body,
    grid=(N_PAD // TC_ROWS,),
    in_specs=[
        pl.BlockSpec((NC, TC_ROWS, KP), lambda i: (0, i, 0)),
        pl.BlockSpec((TC_ROWS, KP), lambda i: (i, 0)),
        pl.BlockSpec((TC_ROWS, KP), lambda i: (i, 0)),
        pl.BlockSpec((NC, TC_ROWS, LANES), lambda i: (0, i, 0)),
        pl.BlockSpec((1, D), lambda i: (0, 0)),
        pl.BlockSpec((D, KP), lambda i: (0, 0)),
    ],
    out_specs=pl.BlockSpec((TC_ROWS, DOM), lambda i: (i, 0)),
    out_shape=jax.ShapeDtypeStruct((N_PAD, DOM), jnp.float32),
)


def kernel(feature, category, edge_index, W, b, domain_memory):
    del category
    ei = edge_index.astype(jnp.int32)
    pad = jnp.full((E_PAD - E,), N, jnp.int32)
    src3 = jnp.concatenate([ei[0], pad]).reshape(NW, NCHUNK, EC)
    dst3 = jnp.concatenate([ei[1], pad]).reshape(NW, NCHUNK, EC)

    feature_pad = jnp.pad(feature, ((0, N_PAD - N), (0, 0)))
    mflat = domain_memory.reshape(K, D)
    mpad = jnp.zeros((D, KP), jnp.float32).at[:, :K].set(mflat.T)
    b_mat = b.reshape(1, D)

    deg_parts = _sc_deg(dst3)
    z, t = _tc_dense(feature_pad, W, mpad, deg_parts)
    y_parts = _sc_agg(src3, dst3, z)
    out_full = _tc_final(y_parts, z, t, deg_parts, b_mat, mpad)
    return out_full[:N, None, :]


# speedup vs baseline: 28.6344x; 1.0196x over previous
import jax
import jax.numpy as jnp
from jax import lax
from jax.experimental import pallas as pl
from jax.experimental.pallas import tpu as pltpu
from jax.experimental.pallas import tpu_sc as plsc

N, E, D, DOM, MEM = 10000, 160000, 256, 4, 10
K = DOM * MEM
KP = 48
NC, NS, LANES = 2, 16, 16
NW = NC * NS
N_PAD = 10240
EC = 128
NCHUNK = 40
E_PAD = NW * NCHUNK * EC
ROWS_S = N_PAD // NS
TC_ROWS = 512
NB = 4

_mesh = plsc.VectorSubcoreMesh(core_axis_name="c", subcore_axis_name="s",
                               num_cores=NC, num_subcores=NS)


def _sc_deg_body(dst_hbm, deg_out, dst_v, ones_v, cbuf, acc):
    cid = lax.axis_index("c")
    sid = lax.axis_index("s")
    wid = sid * NC + cid

    ones16 = jnp.ones((LANES,), jnp.float32)
    zero16 = jnp.zeros((LANES,), jnp.float32)

    def fill_ones(i, carry):
        ones_v[i] = ones16
        return carry

    lax.fori_loop(0, EC, fill_ones, 0)

    def fill_zero(i, carry):
        cbuf[i] = zero16
        return carry

    lax.fori_loop(0, ROWS_S, fill_zero, 0)

    pltpu.sync_copy(cbuf, acc.at[pl.ds(sid * ROWS_S, ROWS_S)])
    plsc.subcore_barrier()

    pltpu.sync_copy(dst_hbm.at[wid], dst_v)

    def step(j, carry):
        pltpu.sync_copy(ones_v, acc.at[dst_v.at[j]], add=True)
        return carry

    lax.fori_loop(0, NCHUNK, step, 0)
    plsc.subcore_barrier()

    pltpu.sync_copy(acc.at[pl.ds(sid * ROWS_S, ROWS_S)], cbuf)
    pltpu.sync_copy(cbuf, deg_out.at[cid, pl.ds(sid * ROWS_S, ROWS_S)])


_sc_deg = pl.kernel(
    _sc_deg_body,
    out_type=jax.ShapeDtypeStruct((NC, N_PAD, LANES), jnp.float32),
    mesh=_mesh,
    scratch_types=[
        pltpu.VMEM((NCHUNK, EC), jnp.int32),
        pltpu.VMEM((EC, LANES), jnp.float32),
        pltpu.VMEM((ROWS_S, LANES), jnp.float32),
        pltpu.VMEM_SHARED((N_PAD, LANES), jnp.float32),
    ],
    compiler_params=pltpu.CompilerParams(use_tc_tiling_on_sc=False),
)


def _sc_agg_body(src_hbm, dst_hbm, z_hbm, y_out, src_v, dst_v, gb0, gb1, gb2,
                 gb3, zbuf, acc, zs, sem0, sem1, sem2, sem3):
    cid = lax.axis_index("c")
    sid = lax.axis_index("s")
    wid = sid * NC + cid
    gbufs = (gb0, gb1, gb2, gb3)
    sems = (sem0, sem1, sem2, sem3)

    zero16 = jnp.zeros((LANES,), jnp.float32)

    def fill_zero(i, carry):
        zbuf[i, pl.ds(0, LANES)] = zero16
        zbuf[i, pl.ds(LANES, LANES)] = zero16
        zbuf[i, pl.ds(2 * LANES, LANES)] = zero16
        return carry

    lax.fori_loop(0, ROWS_S, fill_zero, 0)

    pltpu.sync_copy(zbuf, acc.at[pl.ds(sid * ROWS_S, ROWS_S)])

    pltpu.sync_copy(z_hbm.at[pl.ds(sid * ROWS_S, ROWS_S)], zbuf)
    pltpu.sync_copy(zbuf, zs.at[pl.ds(sid * ROWS_S, ROWS_S)])

    pltpu.sync_copy(src_hbm.at[wid], src_v)
    pltpu.sync_copy(dst_hbm.at[wid], dst_v)
    plsc.subcore_barrier()

    for b in range(NB):
        pltpu.async_copy(zs.at[src_v.at[b]], gbufs[b], sems[b])

    def step(g, carry):
        for b in range(NB):
            j = NB * g + b
            pltpu.make_async_copy(zs.at[src_v.at[j]], gbufs[b],
                                  sems[b]).wait()
            pltpu.sync_copy(gbufs[b], acc.at[dst_v.at[j]], add=True)
            nxt = j + NB

            @pl.when(nxt < NCHUNK)
            def _():
                pltpu.async_copy(zs.at[src_v.at[nxt]], gbufs[b], sems[b])

        return carry

    lax.fori_loop(0, NCHUNK // NB, step, 0)
    plsc.subcore_barrier()

    pltpu.sync_copy(acc.at[pl.ds(sid * ROWS_S, ROWS_S)], zbuf)
    pltpu.sync_copy(zbuf, y_out.at[cid, pl.ds(sid * ROWS_S, ROWS_S)])


_sc_agg = pl.kernel(
    _sc_agg_body,
    out_type=jax.ShapeDtypeStruct((NC, N_PAD, KP), jnp.float32),
    mesh=_mesh,
    scratch_types=[
        pltpu.VMEM((NCHUNK, EC), jnp.int32),
        pltpu.VMEM((NCHUNK, EC), jnp.int32),
        pltpu.VMEM((EC, KP), jnp.float32),
        pltpu.VMEM((EC, KP), jnp.float32),
        pltpu.VMEM((EC, KP), jnp.float32),
        pltpu.VMEM((EC, KP), jnp.float32),
        pltpu.VMEM((ROWS_S, KP), jnp.float32),
        pltpu.VMEM_SHARED((N_PAD, KP), jnp.float32),
        pltpu.VMEM_SHARED((N_PAD, KP), jnp.float32),
        pltpu.SemaphoreType.DMA,
        pltpu.SemaphoreType.DMA,
        pltpu.SemaphoreType.DMA,
        pltpu.SemaphoreType.DMA,
    ],
    compiler_params=pltpu.CompilerParams(use_tc_tiling_on_sc=False),
)


def _tc_dense_body(f_ref, w_ref, m_ref, dp_ref, z_ref, t_ref):
    f = f_ref[...]
    nrm = jnp.sqrt(jnp.sum(f * f, axis=1, keepdims=True))
    feat = f / (nrm + 1e-12)
    h = jnp.dot(feat, w_ref[...], preferred_element_type=jnp.float32)
    hm = jnp.dot(h, m_ref[...], preferred_element_type=jnp.float32)
    tm = jnp.dot(feat, m_ref[...], preferred_element_type=jnp.float32)
    dp = dp_ref[...]
    deg = 1.0 + dp[0, :, 0:1] + dp[1, :, 0:1]
    dinv = lax.rsqrt(deg)
    z_ref[...] = dinv * hm
    lanes = lax.broadcasted_iota(jnp.int32, (TC_ROWS, KP), 1)
    t_ref[...] = jnp.where(lanes == K, dinv, tm)


_tc_dense = pl.pallas_call(
    _tc_dense_body,
    grid=(N_PAD // TC_ROWS,),
    in_specs=[
        pl.BlockSpec((TC_ROWS, D), lambda i: (i, 0)),
        pl.BlockSpec((D, D), lambda i: (0, 0)),
        pl.BlockSpec((D, KP), lambda i: (0, 0)),
        pl.BlockSpec((NC, TC_ROWS, LANES), lambda i: (0, i, 0)),
    ],
    out_specs=[
        pl.BlockSpec((TC_ROWS, KP), lambda i: (i, 0)),
        pl.BlockSpec((TC_ROWS, KP), lambda i: (i, 0)),
    ],
    out_shape=[
        jax.ShapeDtypeStruct((N_PAD, KP), jnp.float32),
        jax.ShapeDtypeStruct((N_PAD, KP), jnp.float32),
    ],
)


def _tc_final_body(yp_ref, z_ref, t_ref, b_ref, m_ref, o_ref):
    rows = lax.broadcasted_iota(jnp.int32, (KP, DOM), 0)
    cols = lax.broadcasted_iota(jnp.int32, (KP, DOM), 1)
    seg = jnp.where((rows // MEM == cols) & (rows < K), 1.0, 0.0)
    rows2 = lax.broadcasted_iota(jnp.int32, (DOM, KP), 0)
    cols2 = lax.broadcasted_iota(jnp.int32, (DOM, KP), 1)
    segt = jnp.where((cols2 // MEM == rows2) & (cols2 < K), 1.0, 0.0)
    mask = (lax.broadcasted_iota(jnp.int32, (1, KP), 1) < K).astype(
        jnp.float32)
    yp = yp_ref[...]
    y = yp[0] + yp[1] + z_ref[...]
    t = t_ref[...]
    dinv = t[:, K:K + 1]
    bm = jnp.dot(b_ref[...], m_ref[...], preferred_element_type=jnp.float32)
    s = dinv * y + bm
    mx4 = jnp.concatenate(
        [jnp.max(s[:, d * MEM:(d + 1) * MEM], axis=1, keepdims=True)
         for d in range(DOM)], axis=1)
    sub = s - jnp.dot(mx4, segt, preferred_element_type=jnp.float32)
    e = jnp.exp(sub) * mask
    den4 = jnp.dot(e, seg, preferred_element_type=jnp.float32)
    num4 = jnp.dot(e * t, seg, preferred_element_type=jnp.float32)
    lg = num4 / den4
    mm = jnp.max(lg, axis=1, keepdims=True)
    ee = jnp.exp(lg - mm)
    o_ref[...] = ee / jnp.sum(ee, axis=1, keepdims=True)


_tc_final = pl.pallas_call(
    _tc_final_body,
    grid=(N_PAD // TC_ROWS,),
    in_specs=[
        pl.BlockSpec((NC, TC_ROWS, KP), lambda i: (0, i, 0)),
        pl.BlockSpec((TC_ROWS, KP), lambda i: (i, 0)),
        pl.BlockSpec((TC_ROWS, KP), lambda i: (i, 0)),
        pl.BlockSpec((1, D), lambda i: (0, 0)),
        pl.BlockSpec((D, KP), lambda i: (0, 0)),
    ],
    out_specs=pl.BlockSpec((TC_ROWS, DOM), lambda i: (i, 0)),
    out_shape=jax.ShapeDtypeStruct((N_PAD, DOM), jnp.float32),
)


def kernel(feature, category, edge_index, W, b, domain_memory):
    del category
    ei = edge_index.astype(jnp.int32)
    pad = jnp.full((E_PAD - E,), N, jnp.int32)
    src3 = jnp.concatenate([ei[0], pad]).reshape(NW, NCHUNK, EC)
    dst3 = jnp.concatenate([ei[1], pad]).reshape(NW, NCHUNK, EC)

    feature_pad = jnp.pad(feature, ((0, N_PAD - N), (0, 0)))
    mflat = domain_memory.reshape(K, D)
    mpad = jnp.zeros((D, KP), jnp.float32).at[:, :K].set(mflat.T)
    b_mat = b.reshape(1, D)

    deg_parts = _sc_deg(dst3)
    z, t = _tc_dense(feature_pad, W, mpad, deg_parts)
    y_parts = _sc_agg(src3, dst3, z)
    out_full = _tc_final(y_parts, z, t, b_mat, mpad)
    return out_full[:N, None, :]


# speedup vs baseline: 31.4875x; 1.0996x over previous
import jax
import jax.numpy as jnp
from jax import lax
from jax.experimental import pallas as pl
from jax.experimental.pallas import tpu as pltpu
from jax.experimental.pallas import tpu_sc as plsc

N, E, D, DOM, MEM = 10000, 160000, 256, 4, 10
K = DOM * MEM
KP = 48
NC, NS, LANES = 2, 16, 16
NW = NC * NS
N_PAD = 10240
EC = 128
NCHUNK = 40
E_PAD = NW * NCHUNK * EC
ROWS_S = N_PAD // NS
TC_ROWS = 1024
NB = 4

_mesh = plsc.VectorSubcoreMesh(core_axis_name="c", subcore_axis_name="s",
                               num_cores=NC, num_subcores=NS)


def _sc_deg_body(dst_hbm, deg_out, dst_v, ones_v, cbuf, acc):
    cid = lax.axis_index("c")
    sid = lax.axis_index("s")
    wid = sid * NC + cid

    ones16 = jnp.ones((LANES,), jnp.float32)
    zero16 = jnp.zeros((LANES,), jnp.float32)

    def fill_ones(i, carry):
        ones_v[i] = ones16
        return carry

    lax.fori_loop(0, EC, fill_ones, 0)

    def fill_zero(i, carry):
        cbuf[i] = zero16
        return carry

    lax.fori_loop(0, ROWS_S, fill_zero, 0)

    pltpu.sync_copy(cbuf, acc.at[pl.ds(sid * ROWS_S, ROWS_S)])
    plsc.subcore_barrier()

    pltpu.sync_copy(dst_hbm.at[wid], dst_v)

    def step(j, carry):
        pltpu.sync_copy(ones_v, acc.at[dst_v.at[j]], add=True)
        return carry

    lax.fori_loop(0, NCHUNK, step, 0)
    plsc.subcore_barrier()

    pltpu.sync_copy(acc.at[pl.ds(sid * ROWS_S, ROWS_S)], cbuf)
    pltpu.sync_copy(cbuf, deg_out.at[cid, pl.ds(sid * ROWS_S, ROWS_S)])


_sc_deg = pl.kernel(
    _sc_deg_body,
    out_type=jax.ShapeDtypeStruct((NC, N_PAD, LANES), jnp.float32),
    mesh=_mesh,
    scratch_types=[
        pltpu.VMEM((NCHUNK, EC), jnp.int32),
        pltpu.VMEM((EC, LANES), jnp.float32),
        pltpu.VMEM((ROWS_S, LANES), jnp.float32),
        pltpu.VMEM_SHARED((N_PAD, LANES), jnp.float32),
    ],
    compiler_params=pltpu.CompilerParams(use_tc_tiling_on_sc=False),
)


def _sc_agg_body(src_hbm, dst_hbm, z_hbm, y_out, src_v, dst_v, gb0, gb1, gb2,
                 gb3, zbuf, acc, zs, sem0, sem1, sem2, sem3):
    cid = lax.axis_index("c")
    sid = lax.axis_index("s")
    wid = sid * NC + cid
    gbufs = (gb0, gb1, gb2, gb3)
    sems = (sem0, sem1, sem2, sem3)

    zero16 = jnp.zeros((LANES,), jnp.float32)

    def fill_zero(i, carry):
        zbuf[i, pl.ds(0, LANES)] = zero16
        zbuf[i, pl.ds(LANES, LANES)] = zero16
        zbuf[i, pl.ds(2 * LANES, LANES)] = zero16
        return carry

    lax.fori_loop(0, ROWS_S, fill_zero, 0)

    pltpu.sync_copy(zbuf, acc.at[pl.ds(sid * ROWS_S, ROWS_S)])

    pltpu.sync_copy(z_hbm.at[pl.ds(sid * ROWS_S, ROWS_S)], zbuf)
    pltpu.sync_copy(zbuf, zs.at[pl.ds(sid * ROWS_S, ROWS_S)])

    pltpu.sync_copy(src_hbm.at[wid], src_v)
    pltpu.sync_copy(dst_hbm.at[wid], dst_v)
    plsc.subcore_barrier()

    for b in range(NB):
        pltpu.async_copy(zs.at[src_v.at[b]], gbufs[b], sems[b])

    def step(g, carry):
        for b in range(NB):
            j = NB * g + b
            pltpu.make_async_copy(zs.at[src_v.at[j]], gbufs[b],
                                  sems[b]).wait()
            pltpu.sync_copy(gbufs[b], acc.at[dst_v.at[j]], add=True)
            nxt = j + NB

            @pl.when(nxt < NCHUNK)
            def _():
                pltpu.async_copy(zs.at[src_v.at[nxt]], gbufs[b], sems[b])

        return carry

    lax.fori_loop(0, NCHUNK // NB, step, 0)
    plsc.subcore_barrier()

    pltpu.sync_copy(acc.at[pl.ds(sid * ROWS_S, ROWS_S)], zbuf)
    pltpu.sync_copy(zbuf, y_out.at[cid, pl.ds(sid * ROWS_S, ROWS_S)])


_sc_agg = pl.kernel(
    _sc_agg_body,
    out_type=jax.ShapeDtypeStruct((NC, N_PAD, KP), jnp.float32),
    mesh=_mesh,
    scratch_types=[
        pltpu.VMEM((NCHUNK, EC), jnp.int32),
        pltpu.VMEM((NCHUNK, EC), jnp.int32),
        pltpu.VMEM((EC, KP), jnp.float32),
        pltpu.VMEM((EC, KP), jnp.float32),
        pltpu.VMEM((EC, KP), jnp.float32),
        pltpu.VMEM((EC, KP), jnp.float32),
        pltpu.VMEM((ROWS_S, KP), jnp.float32),
        pltpu.VMEM_SHARED((N_PAD, KP), jnp.float32),
        pltpu.VMEM_SHARED((N_PAD, KP), jnp.float32),
        pltpu.SemaphoreType.DMA,
        pltpu.SemaphoreType.DMA,
        pltpu.SemaphoreType.DMA,
        pltpu.SemaphoreType.DMA,
    ],
    compiler_params=pltpu.CompilerParams(use_tc_tiling_on_sc=False),
)


def _tc_dense_body(f_ref, w_ref, m_ref, dp_ref, z_ref, t_ref):
    f = f_ref[...]
    nrm = jnp.sqrt(jnp.sum(f * f, axis=1, keepdims=True))
    feat = f / (nrm + 1e-12)
    h = jnp.dot(feat, w_ref[...], preferred_element_type=jnp.float32)
    hm = jnp.dot(h, m_ref[...], preferred_element_type=jnp.float32)
    tm = jnp.dot(feat, m_ref[...], preferred_element_type=jnp.float32)
    dp = dp_ref[...]
    deg = 1.0 + dp[0, :, 0:1] + dp[1, :, 0:1]
    dinv = lax.rsqrt(deg)
    z_ref[...] = dinv * hm
    lanes = lax.broadcasted_iota(jnp.int32, (TC_ROWS, KP), 1)
    t_ref[...] = jnp.where(lanes == K, dinv, tm)


_tc_dense = pl.pallas_call(
    _tc_dense_body,
    grid=(N_PAD // TC_ROWS,),
    in_specs=[
        pl.BlockSpec((TC_ROWS, D), lambda i: (i, 0)),
        pl.BlockSpec((D, D), lambda i: (0, 0)),
        pl.BlockSpec((D, KP), lambda i: (0, 0)),
        pl.BlockSpec((NC, TC_ROWS, LANES), lambda i: (0, i, 0)),
    ],
    out_specs=[
        pl.BlockSpec((TC_ROWS, KP), lambda i: (i, 0)),
        pl.BlockSpec((TC_ROWS, KP), lambda i: (i, 0)),
    ],
    out_shape=[
        jax.ShapeDtypeStruct((N_PAD, KP), jnp.float32),
        jax.ShapeDtypeStruct((N_PAD, KP), jnp.float32),
    ],
)


def _tc_final_body(yp_ref, z_ref, t_ref, b_ref, m_ref, o_ref):
    rows = lax.broadcasted_iota(jnp.int32, (KP, DOM), 0)
    cols = lax.broadcasted_iota(jnp.int32, (KP, DOM), 1)
    seg = jnp.where((rows // MEM == cols) & (rows < K), 1.0, 0.0)
    rows2 = lax.broadcasted_iota(jnp.int32, (DOM, KP), 0)
    cols2 = lax.broadcasted_iota(jnp.int32, (DOM, KP), 1)
    segt = jnp.where((cols2 // MEM == rows2) & (cols2 < K), 1.0, 0.0)
    mask = (lax.broadcasted_iota(jnp.int32, (1, KP), 1) < K).astype(
        jnp.float32)
    yp = yp_ref[...]
    y = yp[0] + yp[1] + z_ref[...]
    t = t_ref[...]
    dinv = t[:, K:K + 1]
    bm = jnp.dot(b_ref[...], m_ref[...], preferred_element_type=jnp.float32)
    s = dinv * y + bm
    mx4 = jnp.concatenate(
        [jnp.max(s[:, d * MEM:(d + 1) * MEM], axis=1, keepdims=True)
         for d in range(DOM)], axis=1)
    sub = s - jnp.dot(mx4, segt, preferred_element_type=jnp.float32)
    e = jnp.exp(sub) * mask
    den4 = jnp.dot(e, seg, preferred_element_type=jnp.float32)
    num4 = jnp.dot(e * t, seg, preferred_element_type=jnp.float32)
    lg = num4 / den4
    mm = jnp.max(lg, axis=1, keepdims=True)
    ee = jnp.exp(lg - mm)
    o_ref[...] = ee / jnp.sum(ee, axis=1, keepdims=True)


_tc_final = pl.pallas_call(
    _tc_final_body,
    grid=(N_PAD // TC_ROWS,),
    in_specs=[
        pl.BlockSpec((NC, TC_ROWS, KP), lambda i: (0, i, 0)),
        pl.BlockSpec((TC_ROWS, KP), lambda i: (i, 0)),
        pl.BlockSpec((TC_ROWS, KP), lambda i: (i, 0)),
        pl.BlockSpec((1, D), lambda i: (0, 0)),
        pl.BlockSpec((D, KP), lambda i: (0, 0)),
    ],
    out_specs=pl.BlockSpec((TC_ROWS, DOM), lambda i: (i, 0)),
    out_shape=jax.ShapeDtypeStruct((N_PAD, DOM), jnp.float32),
)


def kernel(feature, category, edge_index, W, b, domain_memory):
    del category
    ei = edge_index.astype(jnp.int32)
    pad = jnp.full((E_PAD - E,), N, jnp.int32)
    src3 = jnp.concatenate([ei[0], pad]).reshape(NW, NCHUNK, EC)
    dst3 = jnp.concatenate([ei[1], pad]).reshape(NW, NCHUNK, EC)

    feature_pad = jnp.pad(feature, ((0, N_PAD - N), (0, 0)))
    mflat = domain_memory.reshape(K, D)
    mpad = jnp.zeros((D, KP), jnp.float32).at[:, :K].set(mflat.T)
    b_mat = b.reshape(1, D)

    deg_parts = _sc_deg(dst3)
    z, t = _tc_dense(feature_pad, W, mpad, deg_parts)
    y_parts = _sc_agg(src3, dst3, z)
    out_full = _tc_final(y_parts, z, t, b_mat, mpad)
    return out_full[:N, None, :]
